# Initial kernel scaffold; baseline (speedup 1.0000x reference)
import jax
import jax.numpy as jnp
from jax.experimental import pallas as pl


def kernel(embedding, up2down_edge_index, W, b):
    raise NotImplementedError("write your pallas kernel here")


# speedup vs baseline: 29.1232x; 29.1232x over previous
import functools

import jax
import jax.numpy as jnp
from jax import lax
from jax.experimental import pallas as pl
from jax.experimental.pallas import tpu as pltpu
from jax.experimental.pallas import tpu_sc as plsc

N_NODES = 10000
F = 128
N_EDGES = 320000

NC = 2
NS = 16
NW = NC * NS
CHUNK = 128
NCHUNK = 80
EPW = NCHUNK * CHUNK
E_PAD = NW * EPW
N_ACC = 10240
RPT = N_ACC // NS

R = 400
GRID = N_NODES // R

_MESH = plsc.VectorSubcoreMesh(core_axis_name="c", subcore_axis_name="s")


def _deg_body(dst_hbm, deg_out, idx_v, ones_v, zer_v, deg_sh):
    c = lax.axis_index("c")
    s = lax.axis_index("s")
    wid = s * NC + c
    one16 = jnp.ones((16,), jnp.float32)
    zero16 = jnp.zeros((16,), jnp.float32)

    @pl.loop(0, CHUNK // 16)
    def _(i):
        ones_v[pl.ds(i * 16, 16)] = one16

    @pl.loop(0, RPT // 16)
    def _(i):
        zer_v[pl.ds(i * 16, 16)] = zero16

    pltpu.sync_copy(dst_hbm.at[wid], idx_v)
    pltpu.sync_copy(zer_v, deg_sh.at[pl.ds(s * RPT, RPT)])
    plsc.subcore_barrier()

    @pl.loop(0, NCHUNK)
    def _(j):
        pltpu.sync_copy(ones_v, deg_sh.at[idx_v.at[j]], add=True)

    plsc.subcore_barrier()
    pltpu.sync_copy(deg_sh.at[pl.ds(s * RPT, RPT)],
                    deg_out.at[c, pl.ds(s * RPT, RPT)])


_deg_kernel = pl.kernel(
    _deg_body,
    out_type=jax.ShapeDtypeStruct((NC, N_ACC), jnp.float32),
    mesh=_MESH,
    scratch_types=[
        pltpu.VMEM((NCHUNK, CHUNK), jnp.int32),
        pltpu.VMEM((CHUNK,), jnp.float32),
        pltpu.VMEM((RPT,), jnp.float32),
        pltpu.VMEM_SHARED((N_ACC,), jnp.float32),
    ],
)


def _msg_body(src_hbm, dst_hbm, y_hbm, acc_out,
              sidx_v, didx_v, rows_v, zrow_v, acc_sh):
    c = lax.axis_index("c")
    s = lax.axis_index("s")
    wid = s * NC + c
    zero16 = jnp.zeros((16,), jnp.float32)

    @pl.loop(0, 16 * F // 16)
    def _(i):
        zrow_v[i // 8, pl.ds((i % 8) * 16, 16)] = zero16

    pltpu.sync_copy(src_hbm.at[wid], sidx_v)
    pltpu.sync_copy(dst_hbm.at[wid], didx_v)

    @pl.loop(0, RPT // 16)
    def _(k):
        pltpu.sync_copy(zrow_v, acc_sh.at[pl.ds(s * RPT + k * 16, 16), :])

    plsc.subcore_barrier()

    @pl.loop(0, NCHUNK)
    def _(j):
        pltpu.sync_copy(y_hbm.at[sidx_v.at[j]], rows_v)
        pltpu.sync_copy(rows_v, acc_sh.at[didx_v.at[j]], add=True)

    plsc.subcore_barrier()
    pltpu.sync_copy(acc_sh.at[pl.ds(s * RPT, RPT), :],
                    acc_out.at[c, pl.ds(s * RPT, RPT), :])


_msg_kernel = pl.kernel(
    _msg_body,
    out_type=jax.ShapeDtypeStruct((NC, N_ACC, F), jnp.float32),
    mesh=_MESH,
    scratch_types=[
        pltpu.VMEM((NCHUNK, CHUNK), jnp.int32),
        pltpu.VMEM((NCHUNK, CHUNK), jnp.int32),
        pltpu.VMEM((CHUNK, F), jnp.float32),
        pltpu.VMEM((16, F), jnp.float32),
        pltpu.VMEM_SHARED((N_ACC, F), jnp.float32),
    ],
)


def _tc_a_body(emb_ref, w_ref, degp_ref, y_ref, dis_ref):
    deg = degp_ref[0] + degp_ref[1] + 1.0
    dis = lax.rsqrt(deg)
    x = lax.dot_general(emb_ref[...], w_ref[...],
                        (((1,), (1,)), ((), ())),
                        preferred_element_type=jnp.float32)
    dis_ref[...] = dis
    y_ref[...] = x * dis


_tc_a = pl.pallas_call(
    _tc_a_body,
    grid=(GRID,),
    in_specs=[
        pl.BlockSpec((R, F), lambda i: (i, 0)),
        pl.BlockSpec((F, F), lambda i: (0, 0)),
        pl.BlockSpec((NC, R, 1), lambda i: (0, i, 0)),
    ],
    out_specs=[
        pl.BlockSpec((R, F), lambda i: (i, 0)),
        pl.BlockSpec((R, 1), lambda i: (i, 0)),
    ],
    out_shape=[
        jax.ShapeDtypeStruct((N_NODES, F), jnp.float32),
        jax.ShapeDtypeStruct((N_NODES, 1), jnp.float32),
    ],
)


def _tc_b_body(acc_ref, y_ref, dis_ref, b_ref, o_ref):
    o_ref[...] = dis_ref[...] * (acc_ref[0] + acc_ref[1] + y_ref[...]) + b_ref[...]


_tc_b = pl.pallas_call(
    _tc_b_body,
    grid=(GRID,),
    in_specs=[
        pl.BlockSpec((NC, R, F), lambda i: (0, i, 0)),
        pl.BlockSpec((R, F), lambda i: (i, 0)),
        pl.BlockSpec((R, 1), lambda i: (i, 0)),
        pl.BlockSpec((1, F), lambda i: (0, 0)),
    ],
    out_specs=pl.BlockSpec((R, F), lambda i: (i, 0)),
    out_shape=jax.ShapeDtypeStruct((N_NODES, F), jnp.float32),
)


def kernel(embedding, up2down_edge_index, W, b):
    eidx = up2down_edge_index.astype(jnp.int32)
    src, dst = eidx[0], eidx[1]
    npad = E_PAD - src.shape[0]
    ar = jnp.arange(npad, dtype=jnp.int32)
    pad_src = (ar * 131) % N_NODES
    pad_dst = N_NODES + ar % (N_ACC - N_NODES)
    src3 = jnp.concatenate([src, pad_src]).reshape(NW, NCHUNK, CHUNK)
    dst3 = jnp.concatenate([dst, pad_dst]).reshape(NW, NCHUNK, CHUNK)

    deg_parts = _deg_kernel(dst3)
    degp = deg_parts[:, :N_NODES].reshape(NC, N_NODES, 1)
    y, dis = _tc_a(embedding, W, degp)
    acc_parts = _msg_kernel(src3, dst3, y)
    out = _tc_b(acc_parts, y, dis, b.reshape(1, F))
    return out


# speedup vs baseline: 38.9296x; 1.3367x over previous
import functools

import jax
import jax.numpy as jnp
from jax import lax
from jax.experimental import pallas as pl
from jax.experimental.pallas import tpu as pltpu
from jax.experimental.pallas import tpu_sc as plsc

N_NODES = 10000
F = 128
N_EDGES = 320000

NC = 2
NS = 16
NW = NC * NS
CHUNK = 128
NCHUNK = 80
IB = 16
NB = NCHUNK // IB
EPW = NCHUNK * CHUNK
E_PAD = NW * EPW
N_ACC = 10240
RPT = N_ACC // NS

R = 400
GRID = N_NODES // R

_MESH = plsc.VectorSubcoreMesh(core_axis_name="c", subcore_axis_name="s")


def _deg_body(dst_hbm, deg_out, idx_v, ones_v, zer_v, deg_sh):
    c = lax.axis_index("c")
    s = lax.axis_index("s")
    wid = s * NC + c
    one16 = jnp.ones((16,), jnp.float32)
    zero16 = jnp.zeros((16,), jnp.float32)

    @pl.loop(0, CHUNK // 16)
    def _(i):
        ones_v[pl.ds(i * 16, 16)] = one16

    @pl.loop(0, RPT // 16)
    def _(i):
        zer_v[pl.ds(i * 16, 16)] = zero16

    pltpu.sync_copy(dst_hbm.at[wid], idx_v)
    pltpu.sync_copy(zer_v, deg_sh.at[pl.ds(s * RPT, RPT)])
    plsc.subcore_barrier()

    @pl.loop(0, NCHUNK)
    def _(j):
        pltpu.sync_copy(ones_v, deg_sh.at[idx_v.at[j]], add=True)

    plsc.subcore_barrier()
    pltpu.sync_copy(deg_sh.at[pl.ds(s * RPT, RPT)],
                    deg_out.at[c, pl.ds(s * RPT, RPT)])


_deg_kernel = pl.kernel(
    _deg_body,
    out_type=jax.ShapeDtypeStruct((NC, N_ACC), jnp.float32),
    mesh=_MESH,
    scratch_types=[
        pltpu.VMEM((NCHUNK, CHUNK), jnp.int32),
        pltpu.VMEM((CHUNK,), jnp.float32),
        pltpu.VMEM((RPT,), jnp.float32),
        pltpu.VMEM_SHARED((N_ACC,), jnp.float32),
    ],
)


def _msg_body(src_hbm, dst_hbm, y_hbm, acc_out,
              sidx_v, didx_v, rows0_v, rows1_v, zrow_v, acc_sh, sem0, sem1):
    c = lax.axis_index("c")
    s = lax.axis_index("s")
    wid = s * NC + c
    zero16 = jnp.zeros((16,), jnp.float32)

    @pl.loop(0, 16 * F // 16)
    def _(i):
        zrow_v[i // 8, pl.ds((i % 8) * 16, 16)] = zero16

    @pl.loop(0, RPT // 16)
    def _(k):
        pltpu.sync_copy(zrow_v, acc_sh.at[pl.ds(s * RPT + k * 16, 16), :])

    plsc.subcore_barrier()

    pltpu.sync_copy(src_hbm.at[wid, 0], sidx_v)
    pltpu.sync_copy(dst_hbm.at[wid, 0], didx_v)
    pltpu.async_copy(y_hbm.at[sidx_v.at[0]], rows0_v, sem0)

    @pl.loop(0, NCHUNK, step=2)
    def _(j):
        pltpu.async_copy(y_hbm.at[sidx_v.at[(j + 1) % IB]], rows1_v, sem1)
        pltpu.make_async_copy(y_hbm.at[sidx_v.at[j % IB]], rows0_v, sem0).wait()
        pltpu.sync_copy(rows0_v, acc_sh.at[didx_v.at[j % IB]], add=True)

        nxt_block = jnp.logical_and((j + 2) % IB == 0, j + 2 < NCHUNK)

        @pl.when(nxt_block)
        def _():
            pltpu.sync_copy(src_hbm.at[wid, (j + 2) // IB], sidx_v)

        @pl.when(j + 2 < NCHUNK)
        def _():
            pltpu.async_copy(y_hbm.at[sidx_v.at[(j + 2) % IB]], rows0_v, sem0)

        pltpu.make_async_copy(y_hbm.at[sidx_v.at[(j + 1) % IB]], rows1_v,
                              sem1).wait()
        pltpu.sync_copy(rows1_v, acc_sh.at[didx_v.at[(j + 1) % IB]], add=True)

        @pl.when(nxt_block)
        def _():
            pltpu.sync_copy(dst_hbm.at[wid, (j + 2) // IB], didx_v)

    plsc.subcore_barrier()
    pltpu.sync_copy(acc_sh.at[pl.ds(s * RPT, RPT), :],
                    acc_out.at[c, pl.ds(s * RPT, RPT), :])


_msg_kernel = pl.kernel(
    _msg_body,
    out_type=jax.ShapeDtypeStruct((NC, N_ACC, F), jnp.float32),
    mesh=_MESH,
    scratch_types=[
        pltpu.VMEM((IB, CHUNK), jnp.int32),
        pltpu.VMEM((IB, CHUNK), jnp.int32),
        pltpu.VMEM((CHUNK, F), jnp.float32),
        pltpu.VMEM((CHUNK, F), jnp.float32),
        pltpu.VMEM((16, F), jnp.float32),
        pltpu.VMEM_SHARED((N_ACC, F), jnp.float32),
        pltpu.SemaphoreType.DMA,
        pltpu.SemaphoreType.DMA,
    ],
)


def _tc_a_body(emb_ref, w_ref, degp_ref, y_ref, dis_ref):
    deg = degp_ref[0] + degp_ref[1] + 1.0
    dis = lax.rsqrt(deg)
    x = lax.dot_general(emb_ref[...], w_ref[...],
                        (((1,), (1,)), ((), ())),
                        preferred_element_type=jnp.float32)
    dis_ref[...] = dis
    y_ref[...] = x * dis


_tc_a = pl.pallas_call(
    _tc_a_body,
    grid=(GRID,),
    in_specs=[
        pl.BlockSpec((R, F), lambda i: (i, 0)),
        pl.BlockSpec((F, F), lambda i: (0, 0)),
        pl.BlockSpec((NC, R, 1), lambda i: (0, i, 0)),
    ],
    out_specs=[
        pl.BlockSpec((R, F), lambda i: (i, 0)),
        pl.BlockSpec((R, 1), lambda i: (i, 0)),
    ],
    out_shape=[
        jax.ShapeDtypeStruct((N_NODES, F), jnp.float32),
        jax.ShapeDtypeStruct((N_NODES, 1), jnp.float32),
    ],
)


def _tc_b_body(acc_ref, y_ref, dis_ref, b_ref, o_ref):
    o_ref[...] = dis_ref[...] * (acc_ref[0] + acc_ref[1] + y_ref[...]) + b_ref[...]


_tc_b = pl.pallas_call(
    _tc_b_body,
    grid=(GRID,),
    in_specs=[
        pl.BlockSpec((NC, R, F), lambda i: (0, i, 0)),
        pl.BlockSpec((R, F), lambda i: (i, 0)),
        pl.BlockSpec((R, 1), lambda i: (i, 0)),
        pl.BlockSpec((1, F), lambda i: (0, 0)),
    ],
    out_specs=pl.BlockSpec((R, F), lambda i: (i, 0)),
    out_shape=jax.ShapeDtypeStruct((N_NODES, F), jnp.float32),
)


def kernel(embedding, up2down_edge_index, W, b):
    eidx = up2down_edge_index.astype(jnp.int32)
    src, dst = eidx[0], eidx[1]
    npad = E_PAD - src.shape[0]
    ar = jnp.arange(npad, dtype=jnp.int32)
    pad_src = (ar * 131) % N_NODES
    pad_dst = N_NODES + ar % (N_ACC - N_NODES)
    src3 = jnp.concatenate([src, pad_src]).reshape(NW, NCHUNK, CHUNK)
    dst3 = jnp.concatenate([dst, pad_dst]).reshape(NW, NCHUNK, CHUNK)

    src4 = src3.reshape(NW, NB, IB, CHUNK)
    dst4 = dst3.reshape(NW, NB, IB, CHUNK)

    deg_parts = _deg_kernel(dst3)
    degp = deg_parts[:, :N_NODES].reshape(NC, N_NODES, 1)
    y, dis = _tc_a(embedding, W, degp)
    acc_parts = _msg_kernel(src4, dst4, y)
    out = _tc_b(acc_parts, y, dis, b.reshape(1, F))
    return out


# speedup vs baseline: 40.2250x; 1.0333x over previous
import functools

import jax
import jax.numpy as jnp
from jax import lax
from jax.experimental import pallas as pl
from jax.experimental.pallas import tpu as pltpu
from jax.experimental.pallas import tpu_sc as plsc

N_NODES = 10000
F = 128
N_EDGES = 320000

NC = 2
NS = 16
NW = NC * NS
CHUNK = 128
NCHUNK = 80
MC = 64
MNC = 160
MIB = 16
MNB = MNC // MIB
NBUF = 4
EPW = NCHUNK * CHUNK
E_PAD = NW * EPW
N_ACC = 10240
RPT = N_ACC // NS

R = 400
GRID = N_NODES // R

_MESH = plsc.VectorSubcoreMesh(core_axis_name="c", subcore_axis_name="s")


def _deg_body(dst_hbm, deg_out, idx_v, ones_v, zer_v, deg_sh):
    c = lax.axis_index("c")
    s = lax.axis_index("s")
    wid = s * NC + c
    one16 = jnp.ones((16,), jnp.float32)
    zero16 = jnp.zeros((16,), jnp.float32)

    @pl.loop(0, CHUNK // 16)
    def _(i):
        ones_v[pl.ds(i * 16, 16)] = one16

    @pl.loop(0, RPT // 16)
    def _(i):
        zer_v[pl.ds(i * 16, 16)] = zero16

    pltpu.sync_copy(dst_hbm.at[wid], idx_v)
    pltpu.sync_copy(zer_v, deg_sh.at[pl.ds(s * RPT, RPT)])
    plsc.subcore_barrier()

    @pl.loop(0, NCHUNK)
    def _(j):
        pltpu.sync_copy(ones_v, deg_sh.at[idx_v.at[j]], add=True)

    plsc.subcore_barrier()
    pltpu.sync_copy(deg_sh.at[pl.ds(s * RPT, RPT)],
                    deg_out.at[c, pl.ds(s * RPT, RPT)])


_deg_kernel = pl.kernel(
    _deg_body,
    out_type=jax.ShapeDtypeStruct((NC, N_ACC), jnp.float32),
    mesh=_MESH,
    scratch_types=[
        pltpu.VMEM((NCHUNK, CHUNK), jnp.int32),
        pltpu.VMEM((CHUNK,), jnp.float32),
        pltpu.VMEM((RPT,), jnp.float32),
        pltpu.VMEM_SHARED((N_ACC,), jnp.float32),
    ],
)


def _msg_body(src_hbm, dst_hbm, y_hbm, acc_out,
              sidx_v, didx_v, rows_v, zrow_v, acc_sh, sems):
    c = lax.axis_index("c")
    s = lax.axis_index("s")
    wid = s * NC + c
    zero16 = jnp.zeros((16,), jnp.float32)

    @pl.loop(0, 16 * F // 16)
    def _(i):
        zrow_v[i // 8, pl.ds((i % 8) * 16, 16)] = zero16

    @pl.loop(0, RPT // 16)
    def _(k):
        pltpu.sync_copy(zrow_v, acc_sh.at[pl.ds(s * RPT + k * 16, 16), :])

    plsc.subcore_barrier()

    pltpu.sync_copy(src_hbm.at[wid, 0], sidx_v.at[0])
    pltpu.sync_copy(dst_hbm.at[wid, 0], didx_v.at[0])
    for b in range(NBUF):
        pltpu.async_copy(y_hbm.at[sidx_v.at[0, b]], rows_v.at[b], sems[b])

    @pl.loop(0, MNC // NBUF)
    def _(g):
        blk = (g + 3) // 4
        reload = jnp.logical_and(g % 4 == 1, blk < MNB)

        @pl.when(reload)
        def _():
            pltpu.sync_copy(src_hbm.at[wid, blk], sidx_v.at[blk % 2])
            pltpu.sync_copy(dst_hbm.at[wid, blk], didx_v.at[blk % 2])

        for b in range(NBUF):
            cc = g * NBUF + b
            pltpu.make_async_copy(
                y_hbm.at[sidx_v.at[(cc // MIB) % 2, cc % MIB]],
                rows_v.at[b], sems[b]).wait()
            pltpu.sync_copy(
                rows_v.at[b],
                acc_sh.at[didx_v.at[(cc // MIB) % 2, cc % MIB]], add=True)
            nc = cc + NBUF

            @pl.when(nc < MNC)
            def _():
                pltpu.async_copy(
                    y_hbm.at[sidx_v.at[(nc // MIB) % 2, nc % MIB]],
                    rows_v.at[b], sems[b])

    plsc.subcore_barrier()
    pltpu.sync_copy(acc_sh.at[pl.ds(s * RPT, RPT), :],
                    acc_out.at[c, pl.ds(s * RPT, RPT), :])


_msg_kernel = pl.kernel(
    _msg_body,
    out_type=jax.ShapeDtypeStruct((NC, N_ACC, F), jnp.float32),
    mesh=_MESH,
    scratch_types=[
        pltpu.VMEM((2, MIB, MC), jnp.int32),
        pltpu.VMEM((2, MIB, MC), jnp.int32),
        pltpu.VMEM((NBUF, MC, F), jnp.float32),
        pltpu.VMEM((16, F), jnp.float32),
        pltpu.VMEM_SHARED((N_ACC, F), jnp.float32),
        [pltpu.SemaphoreType.DMA] * NBUF,
    ],
)


def _tc_a_body(emb_ref, w_ref, degp_ref, y_ref, dis_ref):
    deg = degp_ref[0] + degp_ref[1] + 1.0
    dis = lax.rsqrt(deg)
    x = lax.dot_general(emb_ref[...], w_ref[...],
                        (((1,), (1,)), ((), ())),
                        preferred_element_type=jnp.float32)
    dis_ref[...] = dis
    y_ref[...] = x * dis


_tc_a = pl.pallas_call(
    _tc_a_body,
    grid=(GRID,),
    in_specs=[
        pl.BlockSpec((R, F), lambda i: (i, 0)),
        pl.BlockSpec((F, F), lambda i: (0, 0)),
        pl.BlockSpec((NC, R, 1), lambda i: (0, i, 0)),
    ],
    out_specs=[
        pl.BlockSpec((R, F), lambda i: (i, 0)),
        pl.BlockSpec((R, 1), lambda i: (i, 0)),
    ],
    out_shape=[
        jax.ShapeDtypeStruct((N_NODES, F), jnp.float32),
        jax.ShapeDtypeStruct((N_NODES, 1), jnp.float32),
    ],
)


def _tc_b_body(acc_ref, y_ref, dis_ref, b_ref, o_ref):
    o_ref[...] = dis_ref[...] * (acc_ref[0] + acc_ref[1] + y_ref[...]) + b_ref[...]


_tc_b = pl.pallas_call(
    _tc_b_body,
    grid=(GRID,),
    in_specs=[
        pl.BlockSpec((NC, R, F), lambda i: (0, i, 0)),
        pl.BlockSpec((R, F), lambda i: (i, 0)),
        pl.BlockSpec((R, 1), lambda i: (i, 0)),
        pl.BlockSpec((1, F), lambda i: (0, 0)),
    ],
    out_specs=pl.BlockSpec((R, F), lambda i: (i, 0)),
    out_shape=jax.ShapeDtypeStruct((N_NODES, F), jnp.float32),
)


def kernel(embedding, up2down_edge_index, W, b):
    eidx = up2down_edge_index.astype(jnp.int32)
    src, dst = eidx[0], eidx[1]
    npad = E_PAD - src.shape[0]
    ar = jnp.arange(npad, dtype=jnp.int32)
    pad_src = (ar * 131) % N_NODES
    pad_dst = N_NODES + ar % (N_ACC - N_NODES)
    src3 = jnp.concatenate([src, pad_src]).reshape(NW, NCHUNK, CHUNK)
    dst3 = jnp.concatenate([dst, pad_dst]).reshape(NW, NCHUNK, CHUNK)

    src4 = src3.reshape(NW, MNB, MIB, MC)
    dst4 = dst3.reshape(NW, MNB, MIB, MC)

    deg_parts = _deg_kernel(dst3)
    degp = deg_parts[:, :N_NODES].reshape(NC, N_NODES, 1)
    y, dis = _tc_a(embedding, W, degp)
    acc_parts = _msg_kernel(src4, dst4, y)
    out = _tc_b(acc_parts, y, dis, b.reshape(1, F))
    return out


# speedup vs baseline: 43.3590x; 1.0779x over previous
import functools

import jax
import jax.numpy as jnp
from jax import lax
from jax.experimental import pallas as pl
from jax.experimental.pallas import tpu as pltpu
from jax.experimental.pallas import tpu_sc as plsc

N_NODES = 10000
F = 128
N_EDGES = 320000

NC = 2
NS = 16
NW = NC * NS
MC = 64
MNC = 160
MIB = 16
MNB = MNC // MIB
NBUF = 4
EPW = MNC * MC
E_PAD = NW * EPW
N_ACC = 10240
RPT = N_ACC // NS

R = 2000
GRID = N_NODES // R

_MESH = plsc.VectorSubcoreMesh(core_axis_name="c", subcore_axis_name="s")


def _deg_body(dst_hbm, deg_out, idx_v, ones_v, zer_v, deg_sh):
    c = lax.axis_index("c")
    s = lax.axis_index("s")
    wid = s * NC + c
    one16 = jnp.ones((16,), jnp.float32)
    zero16 = jnp.zeros((16,), jnp.float32)

    @pl.loop(0, MC // 16)
    def _(i):
        ones_v[pl.ds(i * 16, 16)] = one16

    @pl.loop(0, RPT // 16)
    def _(i):
        zer_v[pl.ds(i * 16, 16)] = zero16

    pltpu.sync_copy(dst_hbm.at[wid], idx_v)
    pltpu.sync_copy(zer_v, deg_sh.at[pl.ds(s * RPT, RPT)])
    plsc.subcore_barrier()

    @pl.loop(0, MNC)
    def _(j):
        pltpu.sync_copy(ones_v, deg_sh.at[idx_v.at[j // MIB, j % MIB]],
                        add=True)

    plsc.subcore_barrier()
    pltpu.sync_copy(deg_sh.at[pl.ds(s * RPT, RPT)],
                    deg_out.at[c, pl.ds(s * RPT, RPT)])


_deg_kernel = pl.kernel(
    _deg_body,
    out_type=jax.ShapeDtypeStruct((NC, N_ACC), jnp.float32),
    mesh=_MESH,
    scratch_types=[
        pltpu.VMEM((MNB, MIB, MC), jnp.int32),
        pltpu.VMEM((MC,), jnp.float32),
        pltpu.VMEM((RPT,), jnp.float32),
        pltpu.VMEM_SHARED((N_ACC,), jnp.float32),
    ],
)


def _msg_body(src_hbm, dst_hbm, y_hbm, acc_out,
              sidx_v, didx_v, rows_v, zrow_v, acc_sh, sems):
    c = lax.axis_index("c")
    s = lax.axis_index("s")
    wid = s * NC + c
    zero16 = jnp.zeros((16,), jnp.float32)

    @pl.loop(0, 16 * F // 16)
    def _(i):
        zrow_v[i // 8, pl.ds((i % 8) * 16, 16)] = zero16

    @pl.loop(0, RPT // 16)
    def _(k):
        pltpu.sync_copy(zrow_v, acc_sh.at[pl.ds(s * RPT + k * 16, 16), :])

    plsc.subcore_barrier()

    pltpu.sync_copy(src_hbm.at[wid, 0], sidx_v.at[0])
    pltpu.sync_copy(dst_hbm.at[wid, 0], didx_v.at[0])
    for b in range(NBUF):
        pltpu.async_copy(y_hbm.at[sidx_v.at[0, b]], rows_v.at[b], sems[b])

    @pl.loop(0, MNC // NBUF)
    def _(g):
        blk = (g + 3) // 4
        reload = jnp.logical_and(g % 4 == 1, blk < MNB)

        @pl.when(reload)
        def _():
            pltpu.sync_copy(src_hbm.at[wid, blk], sidx_v.at[blk % 2])
            pltpu.sync_copy(dst_hbm.at[wid, blk], didx_v.at[blk % 2])

        for b in range(NBUF):
            cc = g * NBUF + b
            pltpu.make_async_copy(
                y_hbm.at[sidx_v.at[(cc // MIB) % 2, cc % MIB]],
                rows_v.at[b], sems[b]).wait()
            pltpu.sync_copy(
                rows_v.at[b],
                acc_sh.at[didx_v.at[(cc // MIB) % 2, cc % MIB]], add=True)
            nc = cc + NBUF

            @pl.when(nc < MNC)
            def _():
                pltpu.async_copy(
                    y_hbm.at[sidx_v.at[(nc // MIB) % 2, nc % MIB]],
                    rows_v.at[b], sems[b])

    plsc.subcore_barrier()
    pltpu.sync_copy(acc_sh.at[pl.ds(s * RPT, RPT), :],
                    acc_out.at[c, pl.ds(s * RPT, RPT), :])


_msg_kernel = pl.kernel(
    _msg_body,
    out_type=jax.ShapeDtypeStruct((NC, N_ACC, F), jnp.float32),
    mesh=_MESH,
    scratch_types=[
        pltpu.VMEM((2, MIB, MC), jnp.int32),
        pltpu.VMEM((2, MIB, MC), jnp.int32),
        pltpu.VMEM((NBUF, MC, F), jnp.float32),
        pltpu.VMEM((16, F), jnp.float32),
        pltpu.VMEM_SHARED((N_ACC, F), jnp.float32),
        [pltpu.SemaphoreType.DMA] * NBUF,
    ],
)


def _tc_a_body(emb_ref, w_ref, degp_ref, y_ref, dis_ref):
    deg = degp_ref[0] + degp_ref[1] + 1.0
    dis = lax.rsqrt(deg)
    x = lax.dot_general(emb_ref[...], w_ref[...],
                        (((1,), (1,)), ((), ())),
                        preferred_element_type=jnp.float32)
    dis_ref[...] = dis
    y_ref[...] = x * dis


_tc_a = pl.pallas_call(
    _tc_a_body,
    grid=(GRID,),
    in_specs=[
        pl.BlockSpec((R, F), lambda i: (i, 0)),
        pl.BlockSpec((F, F), lambda i: (0, 0)),
        pl.BlockSpec((NC, R, 1), lambda i: (0, i, 0)),
    ],
    out_specs=[
        pl.BlockSpec((R, F), lambda i: (i, 0)),
        pl.BlockSpec((R, 1), lambda i: (i, 0)),
    ],
    out_shape=[
        jax.ShapeDtypeStruct((N_NODES, F), jnp.float32),
        jax.ShapeDtypeStruct((N_NODES, 1), jnp.float32),
    ],
)


def _tc_b_body(acc_ref, y_ref, dis_ref, b_ref, o_ref):
    o_ref[...] = dis_ref[...] * (acc_ref[0] + acc_ref[1] + y_ref[...]) + b_ref[...]


_tc_b = pl.pallas_call(
    _tc_b_body,
    grid=(GRID,),
    in_specs=[
        pl.BlockSpec((NC, R, F), lambda i: (0, i, 0)),
        pl.BlockSpec((R, F), lambda i: (i, 0)),
        pl.BlockSpec((R, 1), lambda i: (i, 0)),
        pl.BlockSpec((1, F), lambda i: (0, 0)),
    ],
    out_specs=pl.BlockSpec((R, F), lambda i: (i, 0)),
    out_shape=jax.ShapeDtypeStruct((N_NODES, F), jnp.float32),
)


def kernel(embedding, up2down_edge_index, W, b):
    eidx = up2down_edge_index.astype(jnp.int32)
    src, dst = eidx[0], eidx[1]
    npad = E_PAD - src.shape[0]
    ar = jnp.arange(npad, dtype=jnp.int32)
    pad_src = (ar * 131) % N_NODES
    pad_dst = N_NODES + ar % (N_ACC - N_NODES)
    src4 = jnp.concatenate([src, pad_src]).reshape(NW, MNB, MIB, MC)
    dst4 = jnp.concatenate([dst, pad_dst]).reshape(NW, MNB, MIB, MC)

    deg_parts = _deg_kernel(dst4)
    degp = deg_parts[:, :N_NODES].reshape(NC, N_NODES, 1)
    y, dis = _tc_a(embedding, W, degp)
    acc_parts = _msg_kernel(src4, dst4, y)
    out = _tc_b(acc_parts, y, dis, b.reshape(1, F))
    return out
